# Initial kernel scaffold; baseline (speedup 1.0000x reference)
import jax
import jax.numpy as jnp
from jax.experimental import pallas as pl


def kernel(X, edge_index, edge_attr, W_msg_src, W_msg_edge, b_msg, W_self, W_agg, b_out):
    raise NotImplementedError("write your pallas kernel here")


# speedup vs baseline: 2.9476x; 2.9476x over previous
import functools

import jax
import jax.numpy as jnp
from jax import lax
from jax.experimental import pallas as pl
from jax.experimental.pallas import tpu as pltpu


def _gcn_kernel(nodes, bs, n_edges,
                t_ref, idx_ref, ea_ref, wms_ref, wme_ref, bm_ref, ws_ref,
                wa_ref, bo_ref, out_ref, y_scr, c_scr, agg_scr, cnt_scr):
    f = t_ref.shape[1]

    def y_body(i, _):
        blk = t_ref[pl.ds(i * bs, bs), :]
        y_scr[pl.ds(i * bs, bs), :] = jnp.dot(
            blk, wms_ref[...], preferred_element_type=jnp.float32)
        return 0
    lax.fori_loop(0, nodes, y_body, 0)

    e_chunk = 200
    def c_body(i, _):
        blk = ea_ref[pl.ds(i * e_chunk, e_chunk), :]
        c_scr[pl.ds(i * e_chunk, e_chunk), :] = (
            jnp.dot(blk, wme_ref[...], preferred_element_type=jnp.float32)
            + bm_ref[...])
        return 0
    lax.fori_loop(0, n_edges // e_chunk, c_body, 0)

    def z_body(i, _):
        agg_scr[pl.ds(i * bs, bs), :] = jnp.zeros((bs, f), jnp.float32)
        cnt_scr[i] = 0.0
        return 0
    lax.fori_loop(0, nodes, z_body, 0)

    def e_body(e, _):
        s = idx_ref[0, e]
        d = idx_ref[1, e]
        msg = jnp.maximum(
            y_scr[pl.ds(s * bs, bs), :] + c_scr[pl.ds(e, 1), :], 0.0)
        agg_scr[pl.ds(d * bs, bs), :] += msg
        cnt_scr[d] += 1.0
        return 0
    lax.fori_loop(0, n_edges, e_body, 0)

    def o_body(i, _):
        inv = 1.0 / jnp.maximum(cnt_scr[i], 1.0)
        tblk = t_ref[pl.ds(i * bs, bs), :]
        ablk = agg_scr[pl.ds(i * bs, bs), :] * inv
        h = (jnp.dot(tblk, ws_ref[...], preferred_element_type=jnp.float32)
             + jnp.dot(ablk, wa_ref[...], preferred_element_type=jnp.float32)
             + bo_ref[...])
        out_ref[pl.ds(i * bs, bs), :] = jnp.maximum(h, 0.0)
        return 0
    lax.fori_loop(0, nodes, o_body, 0)


def kernel(X, edge_index, edge_attr, W_msg_src, W_msg_edge, b_msg, W_self,
           W_agg, b_out):
    b, n, s, f_in = X.shape
    bs = b * s
    e = edge_index.shape[1]
    f_out = W_msg_src.shape[1]

    t2d = jnp.transpose(X, (1, 0, 2, 3)).reshape(n * bs, f_in)
    bm2d = b_msg.reshape(1, f_out)
    bo2d = b_out.reshape(1, f_out)

    out2d = pl.pallas_call(
        functools.partial(_gcn_kernel, n, bs, e),
        out_shape=jax.ShapeDtypeStruct((n * bs, f_out), jnp.float32),
        in_specs=[
            pl.BlockSpec(memory_space=pltpu.VMEM),
            pl.BlockSpec(memory_space=pltpu.SMEM),
            pl.BlockSpec(memory_space=pltpu.VMEM),
            pl.BlockSpec(memory_space=pltpu.VMEM),
            pl.BlockSpec(memory_space=pltpu.VMEM),
            pl.BlockSpec(memory_space=pltpu.VMEM),
            pl.BlockSpec(memory_space=pltpu.VMEM),
            pl.BlockSpec(memory_space=pltpu.VMEM),
            pl.BlockSpec(memory_space=pltpu.VMEM),
        ],
        out_specs=pl.BlockSpec(memory_space=pltpu.VMEM),
        scratch_shapes=[
            pltpu.VMEM((n * bs, f_out), jnp.float32),
            pltpu.VMEM((e, f_out), jnp.float32),
            pltpu.VMEM((n * bs, f_out), jnp.float32),
            pltpu.SMEM((n,), jnp.float32),
        ],
    )(t2d, edge_index, edge_attr, W_msg_src, W_msg_edge, bm2d, W_self,
      W_agg, bo2d)

    return jnp.transpose(out2d.reshape(n, b, s, f_out), (1, 0, 2, 3))
